# baseline (speedup 1.0000x reference)
import jax
import jax.numpy as jnp
from jax import lax
from jax.experimental import pallas as pl
from jax.experimental.pallas import tpu as pltpu
from jax.experimental.pallas import tpu_sc as plsc

BATCH = 16384
EMBED = 32
NC = 2
NS = 16
NW = NC * NS
BPW = BATCH // NW
LANES = 16
CHUNK = 64
NCHUNK = BPW // CHUNK


def _sc_body(uidx_hbm, midx_hbm, ut_hbm, ubias_hbm, mt_hbm, mbias_hbm,
             dots_out, ub_out, mb_out,
             uidx_v, midx_v, urows_v, mrows_v, acc_v,
             sem0, sem1, bsem):
    wid = lax.axis_index("s") * NC + lax.axis_index("c")
    base = wid * BPW

    pltpu.sync_copy(uidx_hbm.at[pl.ds(base, BPW)], uidx_v)
    pltpu.sync_copy(midx_hbm.at[pl.ds(base, BPW)], midx_v)

    sems = (sem0, sem1)

    def issue(c, buf):
        def gstep(g, _):
            off = c * CHUNK + g * LANES
            uvec = uidx_v[pl.ds(off, LANES)]
            mvec = midx_v[pl.ds(off, LANES)]
            for jj in range(LANES):
                ui = uvec[jj]
                mi = mvec[jj]
                j = g * LANES + jj
                pltpu.make_async_copy(
                    ut_hbm.at[pl.ds(ui, 1)],
                    urows_v.at[buf].at[pl.ds(j, 1)], sems[buf]).start()
                pltpu.make_async_copy(
                    mt_hbm.at[pl.ds(mi, 1)],
                    mrows_v.at[buf].at[pl.ds(j, 1)], sems[buf]).start()
                pltpu.make_async_copy(
                    ubias_hbm.at[pl.ds(ui, 1)],
                    ub_out.at[pl.ds(base + off + jj, 1)], bsem).start()
                pltpu.make_async_copy(
                    mbias_hbm.at[pl.ds(mi, 1)],
                    mb_out.at[pl.ds(base + off + jj, 1)], bsem).start()
            return 0
        lax.fori_loop(0, CHUNK // LANES, gstep, 0)

    def drain(buf):
        pltpu.make_async_copy(
            ut_hbm.at[pl.ds(0, CHUNK)], urows_v.at[buf], sems[buf]).wait()
        pltpu.make_async_copy(
            mt_hbm.at[pl.ds(0, CHUNK)], mrows_v.at[buf], sems[buf]).wait()

    def compute(buf, accs):
        def row_body(j, accs2):
            b0, b1 = accs2
            b0 = b0 + (urows_v[buf, j, pl.ds(0, LANES)]
                       * mrows_v[buf, j, pl.ds(0, LANES)])
            b1 = b1 + (urows_v[buf, j, pl.ds(LANES, LANES)]
                       * mrows_v[buf, j, pl.ds(LANES, LANES)])
            return (b0, b1)
        return lax.fori_loop(0, CHUNK, row_body, accs)

    zero = jnp.zeros((LANES,), jnp.float32)
    accs = (zero, zero)
    issue(0, 0)
    for c in range(NCHUNK):
        if c + 1 < NCHUNK:
            issue(c + 1, (c + 1) % 2)
        drain(c % 2)
        accs = compute(c % 2, accs)
    a0, a1 = accs
    acc_v[...] = a0 + a1
    pltpu.sync_copy(acc_v, dots_out.at[pl.ds(wid * LANES, LANES)])

    pltpu.make_async_copy(uidx_hbm.at[pl.ds(0, BPW)], uidx_v, bsem).wait()
    pltpu.make_async_copy(midx_hbm.at[pl.ds(0, BPW)], midx_v, bsem).wait()


_sc_stage = pl.kernel(
    _sc_body,
    out_type=(
        jax.ShapeDtypeStruct((NW * LANES,), jnp.float32),
        jax.ShapeDtypeStruct((BATCH, 1), jnp.float32),
        jax.ShapeDtypeStruct((BATCH, 1), jnp.float32),
    ),
    mesh=plsc.VectorSubcoreMesh(
        core_axis_name="c", subcore_axis_name="s", num_cores=NC, num_subcores=NS
    ),
    compiler_params=pltpu.CompilerParams(has_side_effects=True),
    scratch_types=[
        pltpu.VMEM((BPW,), jnp.int32),
        pltpu.VMEM((BPW,), jnp.int32),
        pltpu.VMEM((2, CHUNK, EMBED), jnp.float32),
        pltpu.VMEM((2, CHUNK, EMBED), jnp.float32),
        pltpu.VMEM((LANES,), jnp.float32),
        pltpu.SemaphoreType.DMA,
        pltpu.SemaphoreType.DMA,
        pltpu.SemaphoreType.DMA,
    ],
)


def _tc_body(d_ref, u_ref, m_ref, o_ref):
    s = jnp.sum(d_ref[...])
    x = s + u_ref[...] + m_ref[...]
    o_ref[...] = 1.0 / (1.0 + jnp.exp(-x))


def kernel(inputs, user_table, user_bias_table, movie_table, movie_bias_table):
    user_idx = inputs[:, 0].astype(jnp.int32)
    movie_idx = inputs[:, 1].astype(jnp.int32)

    dots, ub, mb = _sc_stage(
        user_idx, movie_idx, user_table, user_bias_table, movie_table,
        movie_bias_table,
    )

    out = pl.pallas_call(
        _tc_body,
        out_shape=jax.ShapeDtypeStruct((128, 128), jnp.float32),
    )(dots.reshape(4, 128), ub.reshape(128, 128), mb.reshape(128, 128))
    return out.reshape(BATCH, 1)


# speedup vs baseline: 1.2005x; 1.2005x over previous
import jax
import jax.numpy as jnp
from jax import lax
from jax.experimental import pallas as pl
from jax.experimental.pallas import tpu as pltpu

BATCH = 16384
EMBED = 32
CHUNK = 1024
NCHUNK = BATCH // CHUNK


NSTRIPE = 4
STR = CHUNK // NSTRIPE


def _tc_gather_body(uidx_ref, midx_ref, ut, ubt, mt, mbt,
                    sums_ref, ub_o, mb_o,
                    urows, mrows, ubch, mbch, accv,
                    su0, su1, su2, su3, su4, su5, su6, su7,
                    sm0, sm1, sm2, sm3, sm4, sm5, sm6, sm7,
                    sb0, sb1, so0, so1):
    usems = ((su0, su1, su2, su3), (su4, su5, su6, su7))
    msems = ((sm0, sm1, sm2, sm3), (sm4, sm5, sm6, sm7))
    bsems = (sb0, sb1)
    osems = (so0, so1)

    def issue(c, buf):
        cb = c * CHUNK

        def jstep(g, _):
            for k in range(NSTRIPE):
                j = g * NSTRIPE + k
                r = cb + j
                ui = uidx_ref[r]
                mi = midx_ref[r]
                pltpu.make_async_copy(
                    ut.at[pl.ds(ui, 1)],
                    urows.at[buf].at[pl.ds(j, 1)], usems[buf][k]).start()
                pltpu.make_async_copy(
                    mt.at[pl.ds(mi, 1)],
                    mrows.at[buf].at[pl.ds(j, 1)], msems[buf][k]).start()
                pltpu.make_async_copy(
                    ubt.at[pl.ds(ui, 1)],
                    ubch.at[buf].at[pl.ds(j, 1)], bsems[buf]).start()
                pltpu.make_async_copy(
                    mbt.at[pl.ds(mi, 1)],
                    mbch.at[buf].at[pl.ds(j, 1)], bsems[buf]).start()
            return 0
        lax.fori_loop(0, CHUNK // NSTRIPE, jstep, 0)

    def drain_rows(buf):
        for k in range(NSTRIPE):
            pltpu.make_async_copy(
                ut.at[pl.ds(0, STR)],
                urows.at[buf].at[pl.ds(0, STR)], usems[buf][k]).wait()
            pltpu.make_async_copy(
                mt.at[pl.ds(0, STR)],
                mrows.at[buf].at[pl.ds(0, STR)], msems[buf][k]).wait()

    def drain_bias(buf):
        pltpu.make_async_copy(
            ubt.at[pl.ds(0, CHUNK)], ubch.at[buf], bsems[buf]).wait()
        pltpu.make_async_copy(
            mbt.at[pl.ds(0, CHUNK)], mbch.at[buf], bsems[buf]).wait()

    def flush_bias(c, buf):
        cb = c * CHUNK
        pltpu.make_async_copy(
            ubch.at[buf], ub_o.at[pl.ds(cb, CHUNK)], osems[buf]).start()
        pltpu.make_async_copy(
            mbch.at[buf], mb_o.at[pl.ds(cb, CHUNK)], osems[buf]).start()

    accv[...] = jnp.zeros((CHUNK, EMBED), jnp.float32)
    issue(0, 0)
    for c in range(NCHUNK):
        if c + 1 < NCHUNK:
            issue(c + 1, (c + 1) % 2)
        drain_rows(c % 2)
        drain_bias(c % 2)
        flush_bias(c, c % 2)
        accv[...] += urows[c % 2] * mrows[c % 2]
        if c > 0:
            pltpu.make_async_copy(
                ubch.at[(c - 1) % 2],
                ub_o.at[pl.ds((c - 1) * CHUNK, CHUNK)], osems[(c - 1) % 2]).wait()
            pltpu.make_async_copy(
                mbch.at[(c - 1) % 2],
                mb_o.at[pl.ds((c - 1) * CHUNK, CHUNK)], osems[(c - 1) % 2]).wait()

    sums_ref[...] = jnp.full((8, 128), jnp.sum(accv[...]), jnp.float32)
    pltpu.make_async_copy(
        ubch.at[(NCHUNK - 1) % 2],
        ub_o.at[pl.ds((NCHUNK - 1) * CHUNK, CHUNK)], osems[(NCHUNK - 1) % 2]).wait()
    pltpu.make_async_copy(
        mbch.at[(NCHUNK - 1) % 2],
        mb_o.at[pl.ds((NCHUNK - 1) * CHUNK, CHUNK)], osems[(NCHUNK - 1) % 2]).wait()


def _tc_sig_body(d_ref, u_ref, m_ref, o_ref):
    s = d_ref[0, 0]
    o_ref[...] = 1.0 / (1.0 + jnp.exp(-(s + u_ref[...] + m_ref[...])))


def kernel(inputs, user_table, user_bias_table, movie_table, movie_bias_table):
    user_idx = inputs[:, 0].astype(jnp.int32)
    movie_idx = inputs[:, 1].astype(jnp.int32)

    sums, ub, mb = pl.pallas_call(
        _tc_gather_body,
        in_specs=[
            pl.BlockSpec(memory_space=pltpu.SMEM),
            pl.BlockSpec(memory_space=pltpu.SMEM),
            pl.BlockSpec(memory_space=pl.ANY),
            pl.BlockSpec(memory_space=pl.ANY),
            pl.BlockSpec(memory_space=pl.ANY),
            pl.BlockSpec(memory_space=pl.ANY),
        ],
        out_shape=(
            jax.ShapeDtypeStruct((8, 128), jnp.float32),
            jax.ShapeDtypeStruct((BATCH, 1), jnp.float32),
            jax.ShapeDtypeStruct((BATCH, 1), jnp.float32),
        ),
        out_specs=(
            pl.BlockSpec(memory_space=pltpu.VMEM),
            pl.BlockSpec(memory_space=pl.ANY),
            pl.BlockSpec(memory_space=pl.ANY),
        ),
        scratch_shapes=[
            pltpu.VMEM((2, CHUNK, EMBED), jnp.float32),
            pltpu.VMEM((2, CHUNK, EMBED), jnp.float32),
            pltpu.VMEM((2, CHUNK, 1), jnp.float32),
            pltpu.VMEM((2, CHUNK, 1), jnp.float32),
            pltpu.VMEM((CHUNK, EMBED), jnp.float32),
        ] + [pltpu.SemaphoreType.DMA] * 20,
    )(user_idx, movie_idx, user_table, user_bias_table, movie_table,
      movie_bias_table)

    out = pl.pallas_call(
        _tc_sig_body,
        out_shape=jax.ShapeDtypeStruct((128, 128), jnp.float32),
    )(sums, ub.reshape(128, 128), mb.reshape(128, 128))
    return out.reshape(BATCH, 1)


# speedup vs baseline: 1.2011x; 1.0006x over previous
import jax
import jax.numpy as jnp
from jax import lax
from jax.experimental import pallas as pl
from jax.experimental.pallas import tpu as pltpu

BATCH = 16384
EMBED = 32
CHUNK = 1024
NCHUNK = BATCH // CHUNK


def _tc_gather_body(uidx_ref, midx_ref, ut, ubt, mt, mbt,
                    sums_ref, ub_o, mb_o,
                    urows, mrows, ubch, mbch, accv,
                    s0, s1, sb0, sb1, so0, so1):
    sems = (s0, s1)
    bsems = (sb0, sb1)
    osems = (so0, so1)

    def issue(c, buf):
        cb = c * CHUNK

        def jstep(j, _):
            r = cb + j
            ui = uidx_ref[r]
            mi = midx_ref[r]
            pltpu.make_async_copy(
                ut.at[pl.ds(ui, 1)],
                urows.at[buf].at[pl.ds(j, 1)], sems[buf]).start()
            pltpu.make_async_copy(
                mt.at[pl.ds(mi, 1)],
                mrows.at[buf].at[pl.ds(j, 1)], sems[buf]).start()
            pltpu.make_async_copy(
                ubt.at[pl.ds(ui, 1)],
                ubch.at[buf].at[pl.ds(j, 1)], bsems[buf]).start()
            pltpu.make_async_copy(
                mbt.at[pl.ds(mi, 1)],
                mbch.at[buf].at[pl.ds(j, 1)], bsems[buf]).start()
            return 0
        lax.fori_loop(0, CHUNK, jstep, 0)

    def drain_rows(buf):
        pltpu.make_async_copy(
            ut.at[pl.ds(0, CHUNK)], urows.at[buf], sems[buf]).wait()
        pltpu.make_async_copy(
            mt.at[pl.ds(0, CHUNK)], mrows.at[buf], sems[buf]).wait()

    def drain_bias(buf):
        pltpu.make_async_copy(
            ubt.at[pl.ds(0, CHUNK)], ubch.at[buf], bsems[buf]).wait()
        pltpu.make_async_copy(
            mbt.at[pl.ds(0, CHUNK)], mbch.at[buf], bsems[buf]).wait()

    def flush_bias(c, buf):
        cb = c * CHUNK
        pltpu.make_async_copy(
            ubch.at[buf], ub_o.at[pl.ds(cb, CHUNK)], osems[buf]).start()
        pltpu.make_async_copy(
            mbch.at[buf], mb_o.at[pl.ds(cb, CHUNK)], osems[buf]).start()

    accv[...] = jnp.zeros((CHUNK, EMBED), jnp.float32)
    issue(0, 0)
    for c in range(NCHUNK):
        if c + 1 < NCHUNK:
            issue(c + 1, (c + 1) % 2)
        drain_rows(c % 2)
        drain_bias(c % 2)
        flush_bias(c, c % 2)
        accv[...] += urows[c % 2] * mrows[c % 2]
        if c > 0:
            pltpu.make_async_copy(
                ubch.at[(c - 1) % 2],
                ub_o.at[pl.ds((c - 1) * CHUNK, CHUNK)], osems[(c - 1) % 2]).wait()
            pltpu.make_async_copy(
                mbch.at[(c - 1) % 2],
                mb_o.at[pl.ds((c - 1) * CHUNK, CHUNK)], osems[(c - 1) % 2]).wait()

    sums_ref[...] = jnp.full((8, 128), jnp.sum(accv[...]), jnp.float32)
    pltpu.make_async_copy(
        ubch.at[(NCHUNK - 1) % 2],
        ub_o.at[pl.ds((NCHUNK - 1) * CHUNK, CHUNK)], osems[(NCHUNK - 1) % 2]).wait()
    pltpu.make_async_copy(
        mbch.at[(NCHUNK - 1) % 2],
        mb_o.at[pl.ds((NCHUNK - 1) * CHUNK, CHUNK)], osems[(NCHUNK - 1) % 2]).wait()


def _tc_sig_body(d_ref, u_ref, m_ref, o_ref):
    s = d_ref[0, 0]
    o_ref[...] = 1.0 / (1.0 + jnp.exp(-(s + u_ref[...] + m_ref[...])))


def kernel(inputs, user_table, user_bias_table, movie_table, movie_bias_table):
    user_idx = inputs[:, 0].astype(jnp.int32)
    movie_idx = inputs[:, 1].astype(jnp.int32)

    sums, ub, mb = pl.pallas_call(
        _tc_gather_body,
        in_specs=[
            pl.BlockSpec(memory_space=pltpu.SMEM),
            pl.BlockSpec(memory_space=pltpu.SMEM),
            pl.BlockSpec(memory_space=pl.ANY),
            pl.BlockSpec(memory_space=pl.ANY),
            pl.BlockSpec(memory_space=pl.ANY),
            pl.BlockSpec(memory_space=pl.ANY),
        ],
        out_shape=(
            jax.ShapeDtypeStruct((8, 128), jnp.float32),
            jax.ShapeDtypeStruct((BATCH, 1), jnp.float32),
            jax.ShapeDtypeStruct((BATCH, 1), jnp.float32),
        ),
        out_specs=(
            pl.BlockSpec(memory_space=pltpu.VMEM),
            pl.BlockSpec(memory_space=pl.ANY),
            pl.BlockSpec(memory_space=pl.ANY),
        ),
        scratch_shapes=[
            pltpu.VMEM((2, CHUNK, EMBED), jnp.float32),
            pltpu.VMEM((2, CHUNK, EMBED), jnp.float32),
            pltpu.VMEM((2, CHUNK, 1), jnp.float32),
            pltpu.VMEM((2, CHUNK, 1), jnp.float32),
            pltpu.VMEM((CHUNK, EMBED), jnp.float32),
            pltpu.SemaphoreType.DMA,
            pltpu.SemaphoreType.DMA,
            pltpu.SemaphoreType.DMA,
            pltpu.SemaphoreType.DMA,
            pltpu.SemaphoreType.DMA,
            pltpu.SemaphoreType.DMA,
        ],
    )(user_idx, movie_idx, user_table, user_bias_table, movie_table,
      movie_bias_table)

    out = pl.pallas_call(
        _tc_sig_body,
        out_shape=jax.ShapeDtypeStruct((128, 128), jnp.float32),
    )(sums, ub.reshape(128, 128), mb.reshape(128, 128))
    return out.reshape(BATCH, 1)
